# Initial kernel scaffold; baseline (speedup 1.0000x reference)
import jax
import jax.numpy as jnp
from jax.experimental import pallas as pl


def kernel(X, edge_idx, C, B_vec, B_dist):
    raise NotImplementedError("write your pallas kernel here")


# speedup vs baseline: 3.1860x; 3.1860x over previous
import functools

import numpy as np
import jax
import jax.numpy as jnp
from jax import lax
from jax.experimental import pallas as pl
from jax.experimental.pallas import tpu as pltpu
from jax.experimental.pallas import tpu_sc as plsc

_EPS = 1e-3

_RD = np.zeros((8, 64), np.float32)
for _a in range(8):
    for _b in range(8):
        _RD[_b, _a * 8 + _b] += 1.0
        _RD[_a, _a * 8 + _b] -= 1.0


def _sc_gather(table, idx):
    B = idx.shape[0]
    info = plsc.get_sparse_core_info()
    nw = info.num_cores * info.num_subcores
    b_per_w = B // nw
    ch = 2000
    assert b_per_w % ch == 0 and ch % 8 == 0
    n_ch = b_per_w // ch
    mesh = plsc.VectorSubcoreMesh(core_axis_name="c", subcore_axis_name="s")

    @functools.partial(
        pl.kernel,
        mesh=mesh,
        out_type=jax.ShapeDtypeStruct((B, 16), jnp.float32),
        scratch_types=[
            pltpu.VMEM((ch,), jnp.int32),
            pltpu.VMEM((ch, 16), jnp.float32),
            pltpu.SemaphoreType.DMA,
        ],
        compiler_params=pltpu.CompilerParams(use_tc_tiling_on_sc=False),
    )
    def gather_k(table_hbm, idx_hbm, out_hbm, idx_v, rows_v, sem):
        wid = lax.axis_index("s") * info.num_cores + lax.axis_index("c")
        base = wid * b_per_w
        for c in range(n_ch):
            off = base + c * ch
            pltpu.sync_copy(idx_hbm.at[pl.ds(off, ch)], idx_v)
            pltpu.async_copy(table_hbm.at[idx_v], rows_v, sem).wait()
            pltpu.sync_copy(rows_v, out_hbm.at[pl.ds(off, ch)])

    return gather_k(table, idx)


def _cross(a, b):
    a0, a1, a2 = a[:, 0:1], a[:, 1:2], a[:, 2:3]
    b0, b1, b2 = b[:, 0:1], b[:, 1:2], b[:, 2:3]
    return jnp.concatenate(
        [a1 * b2 - a2 * b1, a2 * b0 - a0 * b2, a0 * b1 - a1 * b0], axis=1)


def _nrm(v):
    return v / jnp.sqrt(jnp.sum(v * v, axis=1, keepdims=True) + _EPS)


def _dense_body(x_ref, xj_ref, wv_ref, bd_ref, rd_ref, out_ref):
    rb = x_ref.shape[0]
    e = xj_ref.shape[0]
    k = e // rb
    x = x_ref[...]
    xj = xj_ref[...]
    xn, xca, xc = x[:, 0:3], x[:, 3:6], x[:, 6:9]
    n1 = _nrm(xn - xca)
    u2 = _nrm(xc - xca)
    n2 = _nrm(_cross(n1, u2))
    n3 = _nrm(_cross(n1, n2))
    bund = jnp.concatenate([x[:, 0:12], n1, n2, n3], axis=1)
    bund_e = jnp.broadcast_to(bund[:, None, :], (rb, k, 21)).reshape(e, 21)
    xi = bund_e[:, 0:12]
    n1e = bund_e[:, 12:15]
    n2e = bund_e[:, 15:18]
    n3e = bund_e[:, 18:21]
    def bf(v):
        return v.astype(jnp.bfloat16).astype(jnp.float32)
    dt = bf(xj[:, 3:6] - bund_e[:, 3:6])
    t1 = jnp.sum(bf(n1e) * dt, axis=1, keepdims=True)
    t2 = jnp.sum(bf(n2e) * dt, axis=1, keepdims=True)
    t3 = jnp.sum(bf(n3e) * dt, axis=1, keepdims=True)
    tj16 = jnp.concatenate(
        [t1, t2, t3, jnp.zeros((e, 13), jnp.float32)], axis=1)
    hv = jnp.dot(tj16, wv_ref[...], preferred_element_type=jnp.float32)
    px = jnp.concatenate(
        [xi[:, 0:1], xi[:, 3:4], xi[:, 6:7], xi[:, 9:10],
         xj[:, 0:1], xj[:, 3:4], xj[:, 6:7], xj[:, 9:10]], axis=1)
    py = jnp.concatenate(
        [xi[:, 1:2], xi[:, 4:5], xi[:, 7:8], xi[:, 10:11],
         xj[:, 1:2], xj[:, 4:5], xj[:, 7:8], xj[:, 10:11]], axis=1)
    pz = jnp.concatenate(
        [xi[:, 2:3], xi[:, 5:6], xi[:, 8:9], xi[:, 11:12],
         xj[:, 2:3], xj[:, 5:6], xj[:, 8:9], xj[:, 11:12]], axis=1)
    rd = rd_ref[...]
    hi = lax.Precision.HIGHEST
    dx = jnp.dot(px, rd, precision=hi, preferred_element_type=jnp.float32)
    dy = jnp.dot(py, rd, precision=hi, preferred_element_type=jnp.float32)
    dz = jnp.dot(pz, rd, precision=hi, preferred_element_type=jnp.float32)
    d = jnp.sqrt(dx * dx + dy * dy + dz * dz + _EPS)
    hd = jnp.dot(d, bd_ref[...], preferred_element_type=jnp.float32)
    out_ref[...] = jnp.concatenate(
        [jnp.cos(hv) + jnp.cos(hd), jnp.sin(hv) + jnp.sin(hd)], axis=1)


def _dense(xpad, xje, wv, bd, rd, L, K, rb):
    e = rb * K
    grid = L // rb
    return pl.pallas_call(
        _dense_body,
        grid=(grid,),
        in_specs=[
            pl.BlockSpec((rb, 16), lambda i: (i, 0)),
            pl.BlockSpec((e, 16), lambda i: (i, 0)),
            pl.BlockSpec((16, 64), lambda i: (0, 0)),
            pl.BlockSpec((64, 64), lambda i: (0, 0)),
            pl.BlockSpec((8, 64), lambda i: (0, 0)),
        ],
        out_specs=pl.BlockSpec((e, 128), lambda i: (i, 0)),
        out_shape=jax.ShapeDtypeStruct((L * K, 128), jnp.float32),
    )(xpad, xje, wv, bd, rd)


def kernel(X, edge_idx, C, B_vec, B_dist):
    nb, L, K = edge_idx.shape
    xflat = X.reshape(L, 12)
    xpad = jnp.pad(xflat, ((0, 0), (0, 4)))
    idx = edge_idx.reshape(L * K)
    xje = _sc_gather(xpad, idx)
    wv = jnp.concatenate([B_vec, jnp.zeros((13, 64), jnp.float32)], axis=0)
    rd = jnp.asarray(_RD)
    out = _dense(xpad, xje, wv, B_dist, rd, L, K, rb=80)
    return out.reshape(nb, L, K, 128)
